# baseline (speedup 1.0000x reference)
import functools

import jax
import jax.numpy as jnp
from jax import lax
from jax.experimental import pallas as pl
from jax.experimental.pallas import tpu as pltpu
from jax.experimental.pallas import tpu_sc as plsc


def _gather_kernel(n_total, n_chunks, chunk, idx_hbm, table_hbm,
                   out_hbm, idx_v, rows_v, gsem):
    num_cores = 2
    wid = lax.axis_index("s") * num_cores + lax.axis_index("c")
    per_w = n_total // 32
    base = wid * per_w
    pltpu.sync_copy(idx_hbm.at[pl.ds(base, per_w)], idx_v)

    def body(i, carry):
        off = i * chunk
        cp = pltpu.async_copy(
            table_hbm.at[idx_v.at[pl.ds(off, chunk)]], rows_v, gsem)
        cp.wait()
        pltpu.sync_copy(rows_v, out_hbm.at[pl.ds(base + off, chunk)])
        return carry

    lax.fori_loop(0, n_chunks, body, 0)


def kernel(x, weight):
    batch, fields = x.shape
    vocab, embed = weight.shape
    n_total = batch * fields
    n_workers = 32
    per_w = n_total // n_workers
    chunk = 832
    n_chunks = per_w // chunk

    idx = x.reshape(n_total)

    mesh = plsc.VectorSubcoreMesh(core_axis_name="c", subcore_axis_name="s")
    run = functools.partial(
        pl.kernel,
        mesh=mesh,
        out_type=jax.ShapeDtypeStruct((n_total, embed), jnp.float32),
        scratch_types=[
            pltpu.VMEM((per_w,), jnp.int32),
            pltpu.VMEM((chunk, embed), jnp.float32),
            pltpu.SemaphoreType.DMA,
        ],
        compiler_params=pltpu.CompilerParams(use_tc_tiling_on_sc=False),
    )(functools.partial(_gather_kernel, n_total, n_chunks, chunk))

    out = run(idx, weight)
    return out.reshape(batch, fields, embed)


# speedup vs baseline: 1.0108x; 1.0108x over previous
import functools

import jax
import jax.numpy as jnp
from jax import lax
from jax.experimental import pallas as pl
from jax.experimental.pallas import tpu as pltpu
from jax.experimental.pallas import tpu_sc as plsc


def _gather_kernel(n_total, n_chunks, chunk, idx_hbm, table_hbm,
                   out_hbm, idx_v, rows0, rows1, g0, g1, o0, o1):
    num_cores = 2
    wid = lax.axis_index("s") * num_cores + lax.axis_index("c")
    per_w = n_total // 32
    base = wid * per_w
    pltpu.sync_copy(idx_hbm.at[pl.ds(base, per_w)], idx_v)

    bufs = (rows0, rows1)
    gsems = (g0, g1)
    osems = (o0, o1)

    def start_gather(i):
        b = i % 2
        return pltpu.async_copy(
            table_hbm.at[idx_v.at[pl.ds(i * chunk, chunk)]],
            bufs[b], gsems[b])

    gcps = [start_gather(0), start_gather(1)]
    ocps = [None, None]
    for i in range(n_chunks):
        b = i % 2
        gcps[b].wait()
        ocps[b] = pltpu.async_copy(
            bufs[b], out_hbm.at[pl.ds(base + i * chunk, chunk)], osems[b])
        if i + 2 < n_chunks:
            ocps[b].wait()
            gcps[b] = start_gather(i + 2)
    ocps[0].wait()
    ocps[1].wait()


def kernel(x, weight):
    batch, fields = x.shape
    vocab, embed = weight.shape
    n_total = batch * fields
    n_workers = 32
    per_w = n_total // n_workers
    chunk = 832
    n_chunks = per_w // chunk

    idx = x.reshape(n_total)

    mesh = plsc.VectorSubcoreMesh(core_axis_name="c", subcore_axis_name="s")
    run = functools.partial(
        pl.kernel,
        mesh=mesh,
        out_type=jax.ShapeDtypeStruct((n_total, embed), jnp.float32),
        scratch_types=[
            pltpu.VMEM((per_w,), jnp.int32),
            pltpu.VMEM((chunk, embed), jnp.float32),
            pltpu.VMEM((chunk, embed), jnp.float32),
            pltpu.SemaphoreType.DMA,
            pltpu.SemaphoreType.DMA,
            pltpu.SemaphoreType.DMA,
            pltpu.SemaphoreType.DMA,
        ],
        compiler_params=pltpu.CompilerParams(use_tc_tiling_on_sc=False),
    )(functools.partial(_gather_kernel, n_total, n_chunks, chunk))

    out = run(idx, weight)
    return out.reshape(batch, fields, embed)
